# baseline (speedup 1.0000x reference)
import functools

import jax
import jax.numpy as jnp
from jax import lax
from jax.experimental import pallas as pl
from jax.experimental.pallas import tpu as pltpu
from jax.experimental.pallas import tpu_sc as plsc

N = 10000
E = 320000
D = 128

NC = 2
NS = 16
NW = NC * NS

B = 128
EPT = E // NW
NBUF = 2
NBLK = 80
EPT_PAD = NBLK * B
NPAD = 10240
RPT = NPAD // NS

NH = 2

_mesh = plsc.VectorSubcoreMesh(
    core_axis_name="c", subcore_axis_name="s", num_cores=NC, num_subcores=NS
)


@functools.partial(
    pl.kernel,
    out_type=jax.ShapeDtypeStruct((NC, NPAD, D), jnp.float32),
    mesh=_mesh,
    scratch_types=[
        pltpu.VMEM((NBLK, B), jnp.int32),
        pltpu.VMEM((NBLK, B), jnp.int32),
        pltpu.VMEM((B, D), jnp.float32),
        pltpu.VMEM_SHARED((NPAD, D), jnp.float32),
        pltpu.SemaphoreType.DMA,
    ],
)
def _sc_agg(h_hbm, src_hbm, dst_hbm, out_hbm, src_v, dst_v, r0, acc_sh, sem0):
    c = lax.axis_index("c")
    s = lax.axis_index("s")

    pltpu.sync_copy(src_hbm.at[c, s], src_v)
    pltpu.sync_copy(dst_hbm.at[c, s], dst_v)

    z = jnp.zeros((16,), jnp.float32)

    @pl.loop(0, B)
    def _(r):
        for k in range(D // 16):
            r0[r, pl.ds(k * 16, 16)] = z

    for k in range(RPT // B):
        pltpu.sync_copy(r0, acc_sh.at[pl.ds(s * RPT + k * B, B)])
    plsc.subcore_barrier()

    @pl.loop(0, NBLK)
    def _(j):
        pltpu.async_copy(h_hbm.at[src_v.at[j]], r0, sem0).wait()
        pltpu.sync_copy(r0, acc_sh.at[dst_v.at[j]], add=True)

    plsc.subcore_barrier()
    pltpu.sync_copy(
        acc_sh.at[pl.ds(s * RPT, RPT)], out_hbm.at[c, pl.ds(s * RPT, RPT)]
    )


@functools.partial(
    pl.kernel,
    out_type=jax.ShapeDtypeStruct((NC, NPAD, D), jnp.float32),
    mesh=_mesh,
    scratch_types=[
        pltpu.VMEM((NBLK, B), jnp.int32),
        pltpu.VMEM((B, D), jnp.float32),
        pltpu.VMEM_SHARED((NPAD, D), jnp.float32),
    ],
)
def _sc_deg(dst_hbm, out_hbm, dst_v, ones_v, acc_sh):
    c = lax.axis_index("c")
    s = lax.axis_index("s")

    pltpu.sync_copy(dst_hbm.at[c, s], dst_v)

    z = jnp.zeros((16,), jnp.float32)

    @pl.loop(0, B)
    def _(r):
        for k in range(D // 16):
            ones_v[r, pl.ds(k * 16, 16)] = z

    for k in range(RPT // B):
        pltpu.sync_copy(ones_v, acc_sh.at[pl.ds(s * RPT + k * B, B)])
    plsc.subcore_barrier()

    one = jnp.ones((16,), jnp.float32)

    @pl.loop(0, B)
    def _(r):
        for k in range(D // 16):
            ones_v[r, pl.ds(k * 16, 16)] = one

    @pl.loop(0, NBLK)
    def _(j):
        pltpu.sync_copy(ones_v, acc_sh.at[dst_v.at[j]], add=True)

    plsc.subcore_barrier()
    pltpu.sync_copy(
        acc_sh.at[pl.ds(s * RPT, RPT)], out_hbm.at[c, pl.ds(s * RPT, RPT)]
    )


TC_BLK = 1280


def _tc_post_body(p_ref, dp_ref, w_ref, b_ref, o_ref):
    agg = p_ref[0] + p_ref[1]
    deg = dp_ref[0, :, 0:1] + dp_ref[1, :, 0:1]
    deg = jnp.maximum(deg, 1.0)
    h = agg / deg
    acc = jnp.dot(h, w_ref[...], preferred_element_type=jnp.float32)
    o_ref[...] = jnp.maximum(acc + b_ref[...], 0.0)


def _tc_post(p, degp, w, b):
    return pl.pallas_call(
        _tc_post_body,
        grid=(NPAD // TC_BLK,),
        in_specs=[
            pl.BlockSpec((NC, TC_BLK, D), lambda i: (0, i, 0)),
            pl.BlockSpec((NC, TC_BLK, D), lambda i: (0, i, 0)),
            pl.BlockSpec((D, D), lambda i: (0, 0)),
            pl.BlockSpec((1, D), lambda i: (0, 0)),
        ],
        out_specs=pl.BlockSpec((TC_BLK, D), lambda i: (i, 0)),
        out_shape=jax.ShapeDtypeStruct((NPAD, D), jnp.float32),
    )(p, degp, w, b.reshape(1, D))


def kernel(x, edge_index, W1, b1, W2, b2, W3, b3):
    src = edge_index[0]
    dst = edge_index[1]
    srcb = jnp.pad(src.reshape(NW, EPT), ((0, 0), (0, EPT_PAD - EPT)))
    srcb = srcb.reshape(NC, NS, NBLK, B)
    dstb = jnp.pad(
        dst.reshape(NW, EPT), ((0, 0), (0, EPT_PAD - EPT)), constant_values=N
    )
    dstb = dstb.reshape(NC, NS, NBLK, B)

    degp = _sc_deg(dstb)

    h = jnp.pad(x, ((0, NPAD - N), (0, 0)))
    for w, b in ((W1, b1), (W2, b2), (W3, b3)):
        p = _sc_agg(h, srcb, dstb)
        h = _tc_post(p, degp, w, b)
    return h[:N]


# speedup vs baseline: 1.0051x; 1.0051x over previous
import functools

import jax
import jax.numpy as jnp
from jax import lax
from jax.experimental import pallas as pl
from jax.experimental.pallas import tpu as pltpu
from jax.experimental.pallas import tpu_sc as plsc

N = 10000
E = 320000
D = 128

NC = 2
NS = 16
NW = NC * NS

B = 128
EPT = E // NW
NBUF = 2
NBLK = 80
EPT_PAD = NBLK * B
NPAD = 10240
RPT = NPAD // NS

NH = 2

_mesh = plsc.VectorSubcoreMesh(
    core_axis_name="c", subcore_axis_name="s", num_cores=NC, num_subcores=NS
)


@functools.partial(
    pl.kernel,
    out_type=jax.ShapeDtypeStruct((NC, NPAD, D), jnp.float32),
    mesh=_mesh,
    scratch_types=[
        pltpu.VMEM((NBLK, B), jnp.int32),
        pltpu.VMEM((NBLK, B), jnp.int32),
        pltpu.VMEM((B, D), jnp.float32),
        pltpu.VMEM_SHARED((NPAD, D), jnp.float32),
        pltpu.SemaphoreType.DMA,
    ],
)
def _sc_agg(h_hbm, src_hbm, dst_hbm, out_hbm, src_v, dst_v, r0, acc_sh, sem0):
    c = lax.axis_index("c")
    s = lax.axis_index("s")

    pltpu.sync_copy(src_hbm.at[c, s], src_v)
    pltpu.sync_copy(dst_hbm.at[c, s], dst_v)

    z = jnp.zeros((16,), jnp.float32)

    @pl.loop(0, B)
    def _(r):
        for k in range(D // 16):
            r0[r, pl.ds(k * 16, 16)] = z

    for k in range(RPT // B):
        pltpu.sync_copy(r0, acc_sh.at[pl.ds(s * RPT + k * B, B)])
    plsc.subcore_barrier()

    @pl.loop(0, NBLK)
    def _(j):
        pltpu.async_copy(h_hbm.at[src_v.at[j]], r0, sem0).wait()
        pltpu.sync_copy(r0, acc_sh.at[dst_v.at[j]], add=True)

    plsc.subcore_barrier()
    pltpu.sync_copy(
        acc_sh.at[pl.ds(s * RPT, RPT)], out_hbm.at[c, pl.ds(s * RPT, RPT)]
    )


@functools.partial(
    pl.kernel,
    out_type=jax.ShapeDtypeStruct((NC, NPAD, D), jnp.float32),
    mesh=_mesh,
    scratch_types=[
        pltpu.VMEM((NBLK, B), jnp.int32),
        pltpu.VMEM((B, D), jnp.float32),
        pltpu.VMEM_SHARED((NPAD, D), jnp.float32),
    ],
)
def _sc_deg(dst_hbm, out_hbm, dst_v, ones_v, acc_sh):
    c = lax.axis_index("c")
    s = lax.axis_index("s")

    pltpu.sync_copy(dst_hbm.at[c, s], dst_v)

    z = jnp.zeros((16,), jnp.float32)

    @pl.loop(0, B)
    def _(r):
        for k in range(D // 16):
            ones_v[r, pl.ds(k * 16, 16)] = z

    for k in range(RPT // B):
        pltpu.sync_copy(ones_v, acc_sh.at[pl.ds(s * RPT + k * B, B)])
    plsc.subcore_barrier()

    one = jnp.ones((16,), jnp.float32)

    @pl.loop(0, B)
    def _(r):
        for k in range(D // 16):
            ones_v[r, pl.ds(k * 16, 16)] = one

    @pl.loop(0, NBLK)
    def _(j):
        pltpu.sync_copy(ones_v, acc_sh.at[dst_v.at[j]], add=True)

    plsc.subcore_barrier()
    pltpu.sync_copy(
        acc_sh.at[pl.ds(s * RPT, RPT)], out_hbm.at[c, pl.ds(s * RPT, RPT)]
    )


TC_BLK = 1000


def _tc_post_body(p_ref, dp_ref, w_ref, b_ref, o_ref):
    agg = p_ref[0] + p_ref[1]
    deg = dp_ref[0, :, 0:1] + dp_ref[1, :, 0:1]
    deg = jnp.maximum(deg, 1.0)
    h = agg / deg
    acc = jnp.dot(h, w_ref[...], preferred_element_type=jnp.float32)
    o_ref[...] = jnp.maximum(acc + b_ref[...], 0.0)


def _tc_post(p, degp, w, b):
    return pl.pallas_call(
        _tc_post_body,
        grid=(N // TC_BLK,),
        in_specs=[
            pl.BlockSpec((NC, TC_BLK, D), lambda i: (0, i, 0)),
            pl.BlockSpec((NC, TC_BLK, D), lambda i: (0, i, 0)),
            pl.BlockSpec((D, D), lambda i: (0, 0)),
            pl.BlockSpec((1, D), lambda i: (0, 0)),
        ],
        out_specs=pl.BlockSpec((TC_BLK, D), lambda i: (i, 0)),
        out_shape=jax.ShapeDtypeStruct((N, D), jnp.float32),
    )(p, degp, w, b.reshape(1, D))


def kernel(x, edge_index, W1, b1, W2, b2, W3, b3):
    src = edge_index[0]
    dst = edge_index[1]
    srcb = jnp.pad(src.reshape(NW, EPT), ((0, 0), (0, EPT_PAD - EPT)))
    srcb = srcb.reshape(NC, NS, NBLK, B)
    dstb = jnp.pad(
        dst.reshape(NW, EPT), ((0, 0), (0, EPT_PAD - EPT)), constant_values=N
    )
    dstb = dstb.reshape(NC, NS, NBLK, B)

    degp = _sc_deg(dstb)

    h = x
    for w, b in ((W1, b1), (W2, b2), (W3, b3)):
        p = _sc_agg(h, srcb, dstb)
        h = _tc_post(p, degp, w, b)
    return h


# speedup vs baseline: 1.4550x; 1.4477x over previous
import functools

import jax
import jax.numpy as jnp
from jax import lax
from jax.experimental import pallas as pl
from jax.experimental.pallas import tpu as pltpu
from jax.experimental.pallas import tpu_sc as plsc

N = 10000
E = 320000
D = 128

NC = 2
NS = 16
NW = NC * NS

B = 128
EPT = E // NW
NBUF = 2
NBLK = 79
EPT_PAD = NBLK * B
NPAD = 10240
RPT = NPAD // NS

NH = 2

_mesh = plsc.VectorSubcoreMesh(
    core_axis_name="c", subcore_axis_name="s", num_cores=NC, num_subcores=NS
)


@functools.partial(
    pl.kernel,
    out_type=jax.ShapeDtypeStruct((NC, NPAD, D), jnp.float32),
    mesh=_mesh,
    scratch_types=[
        pltpu.VMEM((NBLK, B), jnp.int32),
        pltpu.VMEM((NBLK, B), jnp.int32),
        pltpu.VMEM((B, D), jnp.float32),
        pltpu.VMEM_SHARED((NPAD, D), jnp.float32),
        pltpu.SemaphoreType.DMA,
    ],
)
def _sc_agg(h_hbm, src_hbm, dst_hbm, out_hbm, src_v, dst_v, r0, acc_sh, sem0):
    c = lax.axis_index("c")
    s = lax.axis_index("s")

    pltpu.sync_copy(src_hbm.at[c, s], src_v)
    pltpu.sync_copy(dst_hbm.at[c, s], dst_v)

    z = jnp.zeros((16,), jnp.float32)

    @pl.loop(0, B)
    def _(r):
        for k in range(D // 16):
            r0[r, pl.ds(k * 16, 16)] = z

    for k in range(RPT // B):
        pltpu.sync_copy(r0, acc_sh.at[pl.ds(s * RPT + k * B, B)])
    plsc.subcore_barrier()

    @pl.loop(0, NBLK)
    def _(j):
        pltpu.async_copy(h_hbm.at[src_v.at[j]], r0, sem0).wait()
        pltpu.sync_copy(r0, acc_sh.at[dst_v.at[j]], add=True)

    plsc.subcore_barrier()
    pltpu.sync_copy(
        acc_sh.at[pl.ds(s * RPT, RPT)], out_hbm.at[c, pl.ds(s * RPT, RPT)]
    )


@functools.partial(
    pl.kernel,
    out_type=jax.ShapeDtypeStruct((NC, NPAD, D), jnp.float32),
    mesh=_mesh,
    scratch_types=[
        pltpu.VMEM((NBLK, B), jnp.int32),
        pltpu.VMEM((B, D), jnp.float32),
        pltpu.VMEM_SHARED((NPAD, D), jnp.float32),
    ],
)
def _sc_deg(dst_hbm, out_hbm, dst_v, ones_v, acc_sh):
    c = lax.axis_index("c")
    s = lax.axis_index("s")

    pltpu.sync_copy(dst_hbm.at[c, s], dst_v)

    z = jnp.zeros((16,), jnp.float32)

    @pl.loop(0, B)
    def _(r):
        for k in range(D // 16):
            ones_v[r, pl.ds(k * 16, 16)] = z

    for k in range(RPT // B):
        pltpu.sync_copy(ones_v, acc_sh.at[pl.ds(s * RPT + k * B, B)])
    plsc.subcore_barrier()

    one = jnp.ones((16,), jnp.float32)

    @pl.loop(0, B)
    def _(r):
        for k in range(D // 16):
            ones_v[r, pl.ds(k * 16, 16)] = one

    @pl.loop(0, NBLK)
    def _(j):
        pltpu.sync_copy(ones_v, acc_sh.at[dst_v.at[j]], add=True)

    plsc.subcore_barrier()
    pltpu.sync_copy(
        acc_sh.at[pl.ds(s * RPT, RPT)], out_hbm.at[c, pl.ds(s * RPT, RPT)]
    )


TC_BLK = 1000


def _tc_post_body(p_ref, dp_ref, w_ref, b_ref, o_ref):
    agg = p_ref[0] + p_ref[1]
    deg = dp_ref[0, :, 0:1] + dp_ref[1, :, 0:1]
    deg = jnp.maximum(deg, 1.0)
    h = agg / deg
    acc = jnp.dot(h, w_ref[...], preferred_element_type=jnp.float32)
    o_ref[...] = jnp.maximum(acc + b_ref[...], 0.0)


def _tc_post(p, degp, w, b):
    return pl.pallas_call(
        _tc_post_body,
        grid=(N // TC_BLK,),
        in_specs=[
            pl.BlockSpec((NC, TC_BLK, D), lambda i: (0, i, 0)),
            pl.BlockSpec((NC, TC_BLK, D), lambda i: (0, i, 0)),
            pl.BlockSpec((D, D), lambda i: (0, 0)),
            pl.BlockSpec((1, D), lambda i: (0, 0)),
        ],
        out_specs=pl.BlockSpec((TC_BLK, D), lambda i: (i, 0)),
        out_shape=jax.ShapeDtypeStruct((N, D), jnp.float32),
    )(p, degp, w, b.reshape(1, D))


def kernel(x, edge_index, W1, b1, W2, b2, W3, b3):
    src = edge_index[0]
    dst = edge_index[1]
    srcb = jnp.pad(src.reshape(NW, EPT), ((0, 0), (0, EPT_PAD - EPT)))
    srcb = srcb.reshape(NC, NS, NBLK, B)
    dstb = jnp.pad(
        dst.reshape(NW, EPT), ((0, 0), (0, EPT_PAD - EPT)), constant_values=N
    )
    dstb = dstb.reshape(NC, NS, NBLK, B)

    degp = _sc_deg(dstb)

    h = x
    for w, b in ((W1, b1), (W2, b2), (W3, b3)):
        p = _sc_agg(h, srcb, dstb)
        h = _tc_post(p, degp, w, b)
    return h
